# baseline (speedup 1.0000x reference)
import functools

import jax
import jax.numpy as jnp
from jax import lax
from jax.experimental import pallas as pl
from jax.experimental.pallas import tpu as pltpu
from jax.experimental.pallas import tpu_sc as plsc

LANES = 16
NC, NS = 2, 16
NW = NC * NS

B, V, L, D = 8, 8, 4096, 128
SEGS = B * V
SEGS_PER_W = SEGS // NW
CHUNK = 256
NCHUNK = L // CHUNK
ROW_UNROLL = 4
DV = D // LANES


def _sc_body(x_hbm, out_hbm, buf0, buf1, outv, sem0, sem1):
    wid = lax.axis_index("s") * NC + lax.axis_index("c")
    base_seg = wid * SEGS_PER_W
    bufs = (buf0, buf1)
    sems = (sem0, sem1)

    def start(g):
        seg = base_seg + (g // NCHUNK)
        row0 = (g % NCHUNK) * CHUNK
        return pltpu.async_copy(
            x_hbm.at[seg, pl.ds(row0, CHUNK)], bufs[g % 2], sems[g % 2]
        )

    total = SEGS_PER_W * NCHUNK
    handle = start(0)
    acc = tuple(jnp.zeros((LANES,), jnp.float32) for _ in range(DV))

    for g in range(total):
        next_handle = start(g + 1) if g + 1 < total else None
        handle.wait()
        buf = bufs[g % 2]

        def body(i, a, buf=buf):
            r = i * ROW_UNROLL
            out = list(a)
            for k in range(ROW_UNROLL):
                for j in range(DV):
                    out[j] = out[j] + buf[r + k, pl.ds(j * LANES, LANES)]
            return tuple(out)

        acc = lax.fori_loop(0, CHUNK // ROW_UNROLL, body, acc)

        if (g + 1) % NCHUNK == 0:
            s = g // NCHUNK
            for j in range(DV):
                v = acc[j] * (1.0 / float(D))
                v = jnp.where(v != v, jnp.zeros((LANES,), jnp.float32), v)
                outv[s, pl.ds(j * LANES, LANES)] = v
            acc = tuple(jnp.zeros((LANES,), jnp.float32) for _ in range(DV))
        handle = next_handle

    pltpu.sync_copy(outv, out_hbm.at[pl.ds(base_seg, SEGS_PER_W)])


@jax.jit
def kernel(X):
    xf = X.reshape(SEGS, L, D)
    out = pl.kernel(
        _sc_body,
        out_type=jax.ShapeDtypeStruct((SEGS, D), jnp.float32),
        mesh=plsc.VectorSubcoreMesh(core_axis_name="c", subcore_axis_name="s"),
        scratch_types=[
            pltpu.VMEM((CHUNK, D), jnp.float32),
            pltpu.VMEM((CHUNK, D), jnp.float32),
            pltpu.VMEM((SEGS_PER_W, D), jnp.float32),
            pltpu.SemaphoreType.DMA,
            pltpu.SemaphoreType.DMA,
        ],
    )(xf)
    return out.reshape(B, V, D)


# speedup vs baseline: 1.2131x; 1.2131x over previous
import functools

import jax
import jax.numpy as jnp
from jax import lax
from jax.experimental import pallas as pl
from jax.experimental.pallas import tpu as pltpu
from jax.experimental.pallas import tpu_sc as plsc

LANES = 16
NC, NS = 2, 16
NW = NC * NS

B, V, L, D = 8, 8, 4096, 128
SEGS = B * V
SEGS_PER_W = SEGS // NW

L_SC = 1536
L_TC = L - L_SC

CHUNK = 256
NCHUNK = L_SC // CHUNK
ROW_UNROLL = 4
DV = D // LANES

TC_BLK = 512
TC_NBLK = L_TC // TC_BLK
TC_SEGBLK = 8


def _sc_body(x_hbm, out_hbm, buf0, buf1, outv, sem0, sem1):
    wid = lax.axis_index("s") * NC + lax.axis_index("c")
    base_seg = wid * SEGS_PER_W
    bufs = (buf0, buf1)
    sems = (sem0, sem1)

    def start(g):
        seg = base_seg + (g // NCHUNK)
        row0 = (g % NCHUNK) * CHUNK
        return pltpu.async_copy(
            x_hbm.at[seg, pl.ds(row0, CHUNK)], bufs[g % 2], sems[g % 2]
        )

    total = SEGS_PER_W * NCHUNK
    handle = start(0)
    acc = tuple(jnp.zeros((LANES,), jnp.float32) for _ in range(DV))

    for g in range(total):
        next_handle = start(g + 1) if g + 1 < total else None
        handle.wait()
        buf = bufs[g % 2]

        def body(i, a, buf=buf):
            r = i * ROW_UNROLL
            out = list(a)
            for k in range(ROW_UNROLL):
                for j in range(DV):
                    out[j] = out[j] + buf[r + k, pl.ds(j * LANES, LANES)]
            return tuple(out)

        acc = lax.fori_loop(0, CHUNK // ROW_UNROLL, body, acc)

        if (g + 1) % NCHUNK == 0:
            s = g // NCHUNK
            for j in range(DV):
                outv[s, pl.ds(j * LANES, LANES)] = acc[j] * (1.0 / float(D))
            acc = tuple(jnp.zeros((LANES,), jnp.float32) for _ in range(DV))
        handle = next_handle

    pltpu.sync_copy(outv, out_hbm.at[pl.ds(base_seg, SEGS_PER_W)])


TC_BLK0 = L_SC // TC_BLK


def _tc_body(x_ref, o_ref):
    j = pl.program_id(1)

    @pl.when(j == 0)
    def _():
        o_ref[...] = jnp.zeros_like(o_ref)

    o_ref[...] += jnp.sum(x_ref[...], axis=1) * (1.0 / float(D))


@jax.jit
def kernel(X):
    xf = X.reshape(SEGS, L, D)

    sc_part = pl.kernel(
        _sc_body,
        out_type=jax.ShapeDtypeStruct((SEGS, D), jnp.float32),
        mesh=plsc.VectorSubcoreMesh(core_axis_name="c", subcore_axis_name="s"),
        scratch_types=[
            pltpu.VMEM((CHUNK, D), jnp.float32),
            pltpu.VMEM((CHUNK, D), jnp.float32),
            pltpu.VMEM((SEGS_PER_W, D), jnp.float32),
            pltpu.SemaphoreType.DMA,
            pltpu.SemaphoreType.DMA,
        ],
    )(xf)

    tc_part = pl.pallas_call(
        _tc_body,
        grid=(SEGS // TC_SEGBLK, TC_NBLK),
        in_specs=[
            pl.BlockSpec(
                (TC_SEGBLK, TC_BLK, D), lambda i, j: (i, j + TC_BLK0, 0)
            )
        ],
        out_specs=pl.BlockSpec((TC_SEGBLK, D), lambda i, j: (i, 0)),
        out_shape=jax.ShapeDtypeStruct((SEGS, D), jnp.float32),
    )(xf)

    ret = sc_part + tc_part
    ret = jnp.where(jnp.isnan(ret), jnp.zeros_like(ret), ret)
    return ret.reshape(B, V, D)
